# baseline (speedup 1.0000x reference)
import functools

import jax
import jax.numpy as jnp
import numpy as np
from jax import lax
from jax.experimental import pallas as pl
from jax.experimental.pallas import tpu as pltpu
from jax.experimental.pallas import tpu_sc as plsc

B = 4096
ND = 13
NF = 26
V = 100000
ED = 128
NFI = NF + 1
PAD = 32

_NSPLIT = 2
_BB = 512
_WIN_SAMPLES = 8
_WIN = PAD * _WIN_SAMPLES


def _sc_gather(emb, idx32):
    n = idx32.shape[0]
    idx2 = idx32.reshape(1, n)
    mesh = plsc.VectorSubcoreMesh(core_axis_name="c", subcore_axis_name="s")

    @functools.partial(
        pl.kernel,
        out_type=jax.ShapeDtypeStruct((n, ED), emb.dtype),
        mesh=mesh,
    )
    def gather_kernel(emb_hbm, idx_hbm, out_hbm):
        def body(i_vmem, o_vmem):
            pltpu.sync_copy(emb_hbm.at[i_vmem.at[0]], o_vmem)

        pltpu.emit_pipeline(
            body,
            grid=(n // _WIN,),
            in_specs=[
                pl.BlockSpec((1, _WIN), index_map=lambda i: (0, i))
            ],
            out_specs=[
                pl.BlockSpec((_WIN, ED), index_map=lambda i: (i, 0))
            ],
            core_axis_name=("c", "s"),
            dimension_semantics=(pltpu.PARALLEL,),
        )(idx_hbm, out_hbm)

    return gather_kernel(emb, idx2)


def _tc_body(x_ref, embf_ref, bw0_ref, bb0_ref, bw1_ref, bb1_ref, bw2_ref,
             bb2_ref, tw0a_ref, wfull_ref, tb0_ref, tw1_ref, tb1_ref,
             tw2_ref, tb2_ref, tw3_ref, tb3_ref, tw4_ref, tb4_ref, out_ref):
    f32 = jnp.float32
    dense = x_ref[:, :ND]
    h = jnp.maximum(jnp.dot(dense, bw0_ref[...], preferred_element_type=f32)
                    + bb0_ref[...], 0.0)
    h = jnp.maximum(jnp.dot(h, bw1_ref[...], preferred_element_type=f32)
                    + bb1_ref[...], 0.0)
    bot = jnp.maximum(jnp.dot(h, bw2_ref[...], preferred_element_type=f32)
                      + bb2_ref[...], 0.0)

    embf_ref[:, NF:NF + 1, :] = bot[:, None, :]
    s = embf_ref[:, :NFI, :]
    xact = lax.dot_general(s, s, (((2,), (2,)), ((0,), (0,))),
                           preferred_element_type=f32)
    xflat = xact.reshape(_BB, NFI * NFI)

    h = (jnp.dot(bot, tw0a_ref[...], preferred_element_type=f32)
         + jnp.dot(xflat, wfull_ref[...], preferred_element_type=f32)
         + tb0_ref[...])
    h = jnp.maximum(h, 0.0)
    h = jnp.maximum(jnp.dot(h, tw1_ref[...], preferred_element_type=f32)
                    + tb1_ref[...], 0.0)
    h = jnp.maximum(jnp.dot(h, tw2_ref[...], preferred_element_type=f32)
                    + tb2_ref[...], 0.0)
    h = jnp.maximum(jnp.dot(h, tw3_ref[...], preferred_element_type=f32)
                    + tb3_ref[...], 0.0)
    out_ref[...] = (jnp.dot(h, tw4_ref[...], preferred_element_type=f32)
                    + tb4_ref[...])


def _full_spec(shape):
    nd = len(shape)
    return pl.BlockSpec(shape, lambda i, _nd=nd: (0,) * _nd)


_TRIU_I, _TRIU_J = np.triu_indices(NFI)
_PAIR_POS = np.zeros((NFI, NFI), dtype=np.int32)
_PAIR_POS[_TRIU_I, _TRIU_J] = np.arange(_TRIU_I.shape[0], dtype=np.int32)
_ORD = np.array([i + 1 for i in range(NF)] + [0], dtype=np.int32)
_P, _Q = np.meshgrid(np.arange(NFI), np.arange(NFI), indexing="ij")
_OI, _OJ = _ORD[_P], _ORD[_Q]
_POS = _PAIR_POS[np.minimum(_OI, _OJ), np.maximum(_OI, _OJ)]
_POS_FLAT = _POS.reshape(-1)
_MASK_FLAT = (_P <= _Q).astype(np.float32).reshape(-1)


def kernel(x, train, bw0, bb0, bw1, bb1, bw2, bb2, emb, tw0, tb0, tw1, tb1,
           tw2, tb2, tw3, tb3, tw4, tb4):
    del train
    cat = x[:, ND:].astype(jnp.int32)
    idxm = cat + (jnp.arange(NF, dtype=jnp.int32) * V)[None, :]
    idx32 = jnp.concatenate(
        [idxm, jnp.zeros((B, PAD - NF), jnp.int32)], axis=1).reshape(-1)

    tw0a = tw0[:ED]
    wtri = tw0[ED:]
    wfull = jnp.take(wtri, _POS_FLAT, axis=0) * _MASK_FLAT[:, None]

    bb0, bb1, bb2, tb0, tb1, tb2, tb3, tb4 = (
        b.reshape(1, -1) for b in (bb0, bb1, bb2, tb0, tb1, tb2, tb3, tb4))

    bh = B // _NSPLIT
    embfs = [
        _sc_gather(emb, lax.dynamic_slice_in_dim(idx32, h * bh * PAD, bh * PAD))
        .reshape(bh, PAD, ED)
        for h in range(_NSPLIT)
    ]

    grid = (bh // _BB,)
    outs = []
    for h in range(_NSPLIT):
        xh = lax.dynamic_slice_in_dim(x, h * bh, bh)
        outs.append(pl.pallas_call(
            _tc_body,
            grid=grid,
            in_specs=[
                pl.BlockSpec((_BB, ND + NF), lambda i: (i, 0)),
                pl.BlockSpec((_BB, PAD, ED), lambda i: (i, 0, 0)),
                _full_spec(bw0.shape), _full_spec((1, 512)),
                _full_spec(bw1.shape), _full_spec((1, 256)),
                _full_spec(bw2.shape), _full_spec((1, 128)),
                _full_spec(tw0a.shape), _full_spec((NFI * NFI, 1024)),
                _full_spec((1, 1024)),
                _full_spec(tw1.shape), _full_spec((1, 1024)),
                _full_spec(tw2.shape), _full_spec((1, 512)),
                _full_spec(tw3.shape), _full_spec((1, 256)),
                _full_spec(tw4.shape), _full_spec((1, 1)),
            ],
            out_specs=pl.BlockSpec((_BB, 1), lambda i: (i, 0)),
            out_shape=jax.ShapeDtypeStruct((bh, 1), jnp.float32),
        )(xh, embfs[h], bw0, bb0, bw1, bb1, bw2, bb2, tw0a, wfull, tb0,
          tw1, tb1, tw2, tb2, tw3, tb3, tw4, tb4))
    return jnp.concatenate(outs, axis=0)


# speedup vs baseline: 8.5630x; 8.5630x over previous
import functools

import jax
import jax.numpy as jnp
import numpy as np
from jax import lax
from jax.experimental import pallas as pl
from jax.experimental.pallas import tpu as pltpu
from jax.experimental.pallas import tpu_sc as plsc

B = 4096
ND = 13
NF = 26
V = 100000
ED = 128
NFI = NF + 1
NIDX = B * NF


_GATHER_WINDOW = 256
_NSPLIT = 2


def _sc_gather(emb, idx):
    n = idx.shape[0]
    idx2 = idx.reshape(1, n)
    mesh = plsc.VectorSubcoreMesh(core_axis_name="c", subcore_axis_name="s")

    @functools.partial(
        pl.kernel,
        out_type=jax.ShapeDtypeStruct((n, ED), emb.dtype),
        mesh=mesh,
    )
    def gather_kernel(emb_hbm, idx_hbm, out_hbm):
        def body(i_vmem, o_vmem):
            pltpu.sync_copy(emb_hbm.at[i_vmem.at[0]], o_vmem)

        pltpu.emit_pipeline(
            body,
            grid=(n // _GATHER_WINDOW,),
            in_specs=[
                pl.BlockSpec((1, _GATHER_WINDOW), index_map=lambda i: (0, i))
            ],
            out_specs=[
                pl.BlockSpec((_GATHER_WINDOW, ED), index_map=lambda i: (i, 0))
            ],
            core_axis_name=("c", "s"),
            dimension_semantics=(pltpu.PARALLEL,),
        )(idx_hbm, out_hbm)

    return gather_kernel(emb, idx2)


_BB = 512


def _tc_body(x_ref, embf_ref, bw0_ref, bb0_ref, bw1_ref, bb1_ref, bw2_ref,
             bb2_ref, tw0a_ref, wfull_ref, tb0_ref, tw1_ref, tb1_ref,
             tw2_ref, tb2_ref, tw3_ref, tb3_ref, tw4_ref, tb4_ref, out_ref):
    f32 = jnp.float32
    dense = x_ref[:, :ND]
    h = jnp.maximum(jnp.dot(dense, bw0_ref[...], preferred_element_type=f32)
                    + bb0_ref[...], 0.0)
    h = jnp.maximum(jnp.dot(h, bw1_ref[...], preferred_element_type=f32)
                    + bb1_ref[...], 0.0)
    bot = jnp.maximum(jnp.dot(h, bw2_ref[...], preferred_element_type=f32)
                      + bb2_ref[...], 0.0)

    s_emb = embf_ref[...].reshape(_BB, NF, ED)
    s = jnp.concatenate([bot[:, None, :], s_emb], axis=1)
    xact = lax.dot_general(s, s, (((2,), (2,)), ((0,), (0,))),
                           preferred_element_type=f32)
    xflat = xact.reshape(_BB, NFI * NFI)

    h = (jnp.dot(bot, tw0a_ref[...], preferred_element_type=f32)
         + jnp.dot(xflat, wfull_ref[...], preferred_element_type=f32)
         + tb0_ref[...])
    h = jnp.maximum(h, 0.0)
    h = jnp.maximum(jnp.dot(h, tw1_ref[...], preferred_element_type=f32)
                    + tb1_ref[...], 0.0)
    h = jnp.maximum(jnp.dot(h, tw2_ref[...], preferred_element_type=f32)
                    + tb2_ref[...], 0.0)
    h = jnp.maximum(jnp.dot(h, tw3_ref[...], preferred_element_type=f32)
                    + tb3_ref[...], 0.0)
    out_ref[...] = (jnp.dot(h, tw4_ref[...], preferred_element_type=f32)
                    + tb4_ref[...])


def _full_spec(shape):
    nd = len(shape)
    return pl.BlockSpec(shape, lambda i, _nd=nd: (0,) * _nd)


_TRIU_I, _TRIU_J = np.triu_indices(NFI)
_PAIR_POS = np.zeros((NFI, NFI), dtype=np.int32)
_PAIR_POS[_TRIU_I, _TRIU_J] = np.arange(_TRIU_I.shape[0], dtype=np.int32)
_PAIR_MASK = np.triu(np.ones((NFI, NFI), dtype=np.float32))
_PAIR_POS_FLAT = _PAIR_POS.reshape(-1)
_PAIR_MASK_FLAT = _PAIR_MASK.reshape(-1)


def kernel(x, train, bw0, bb0, bw1, bb1, bw2, bb2, emb, tw0, tb0, tw1, tb1,
           tw2, tb2, tw3, tb3, tw4, tb4):
    del train
    cat = x[:, ND:].astype(jnp.int32)
    idx = (cat + (jnp.arange(NF, dtype=jnp.int32) * V)[None, :]).reshape(-1)

    tw0a = tw0[:ED]
    wtri = tw0[ED:]
    wfull = jnp.take(wtri, _PAIR_POS_FLAT, axis=0) * _PAIR_MASK_FLAT[:, None]

    bb0, bb1, bb2, tb0, tb1, tb2, tb3, tb4 = (
        b.reshape(1, -1) for b in (bb0, bb1, bb2, tb0, tb1, tb2, tb3, tb4))

    bh = B // _NSPLIT
    embfs = [_sc_gather(emb, lax.dynamic_slice_in_dim(idx, h * bh * NF, bh * NF))
             for h in range(_NSPLIT)]

    grid = (bh // _BB,)
    outs = []
    for h in range(_NSPLIT):
        xh = lax.dynamic_slice_in_dim(x, h * bh, bh)
        outs.append(pl.pallas_call(
            _tc_body,
            grid=grid,
            in_specs=[
                pl.BlockSpec((_BB, ND + NF), lambda i: (i, 0)),
                pl.BlockSpec((_BB * NF, ED), lambda i: (i, 0)),
                _full_spec(bw0.shape), _full_spec(bb0.shape),
                _full_spec(bw1.shape), _full_spec(bb1.shape),
                _full_spec(bw2.shape), _full_spec(bb2.shape),
                _full_spec(tw0a.shape), _full_spec(wfull.shape),
                _full_spec(tb0.shape),
                _full_spec(tw1.shape), _full_spec(tb1.shape),
                _full_spec(tw2.shape), _full_spec(tb2.shape),
                _full_spec(tw3.shape), _full_spec(tb3.shape),
                _full_spec(tw4.shape), _full_spec(tb4.shape),
            ],
            out_specs=pl.BlockSpec((_BB, 1), lambda i: (i, 0)),
            out_shape=jax.ShapeDtypeStruct((bh, 1), jnp.float32),
        )(xh, embfs[h], bw0, bb0, bw1, bb1, bw2, bb2, tw0a, wfull, tb0,
          tw1, tb1, tw2, tb2, tw3, tb3, tw4, tb4))
    return jnp.concatenate(outs, axis=0)
